# baseline (speedup 1.0000x reference)
import dataclasses
import functools

import jax
import jax.numpy as jnp
from jax import lax
from jax.experimental import pallas as pl
from jax.experimental.pallas import tpu as pltpu
from jax.experimental.pallas import tpu_sc as plsc

N_CORES = 2
N_SUBCORES = 16
N_WORKERS = N_CORES * N_SUBCORES
LANES = 16
C_PER_TILE = 4

EDGE_CHUNK = 16000
NBUF = 2
GROUP_UNROLL = 4


def _scatter_accumulate(xt, edges, n_nodes, n_chunks):
    mesh = plsc.VectorSubcoreMesh(core_axis_name="c", subcore_axis_name="s")
    cp = pltpu.CompilerParams()
    if "needs_layout_passes" in pltpu.CompilerParams.__dataclass_fields__:
        cp = dataclasses.replace(cp, needs_layout_passes=False)

    n_pairs = C_PER_TILE // 2

    @functools.partial(
        pl.kernel,
        compiler_params=cp,
        out_type=jax.ShapeDtypeStruct((N_WORKERS * C_PER_TILE, n_nodes), jnp.float32),
        mesh=mesh,
        scratch_types=(
            [pltpu.VMEM((n_nodes,), jnp.int32) for _ in range(n_pairs)]
            + [pltpu.VMEM((n_nodes,), jnp.float32) for _ in range(C_PER_TILE)]
            + [
                pltpu.VMEM((EDGE_CHUNK + EDGE_CHUNK // 2,), jnp.int32),
                pltpu.VMEM((EDGE_CHUNK + EDGE_CHUNK // 2,), jnp.int32),
                pltpu.SemaphoreType.DMA,
                pltpu.SemaphoreType.DMA,
            ]
        ),
    )
    def sc_kernel(xt_hbm, e_hbm, o_hbm, xp0, xp1, a0, a1, a2, a3,
                  ebuf0, ebuf1, sem0, sem1):
        ebufs = (ebuf0, ebuf1)
        xps = (xp0, xp1)
        accs = (a0, a1, a2, a3)
        sems = (sem0, sem1)
        wid = lax.axis_index("c") * N_SUBCORES + lax.axis_index("s")
        pair0 = wid * n_pairs

        for j in range(n_pairs):
            pltpu.sync_copy(xt_hbm.at[pair0 + j], xps[j])

        zeros = jnp.zeros((LANES,), jnp.float32)

        @pl.loop(0, n_nodes, step=LANES)
        def _(i):
            for j in range(C_PER_TILE):
                accs[j][pl.ds(i, LANES)] = zeros

        for b in range(NBUF):
            pltpu.make_async_copy(e_hbm.at[b], ebufs[b], sems[b]).start()

        himask = jnp.int32(-65536)

        def process(buf):
            half = EDGE_CHUNK // 2

            @plsc.parallel_loop(0, half, step=LANES, unroll=3)
            def _(i):
                wp = buf[pl.ds(EDGE_CHUNK + i, LANES)]
                wa = plsc.bitcast(wp << 16, jnp.float32)
                wb = plsc.bitcast(wp & himask, jnp.float32)
                for pk, wv in (
                    (buf[pl.ds(i, LANES)], wa),
                    (buf[pl.ds(i + half, LANES)], wb),
                ):
                    s = pk >> 14
                    d = pk & 16383
                    for j in range(n_pairs):
                        g = plsc.load_gather(xps[j], [s])
                        vlo = plsc.bitcast(g << 16, jnp.float32)
                        vhi = plsc.bitcast(g & himask, jnp.float32)
                        plsc.addupdate_scatter(accs[2 * j], [d], vlo * wv)
                        plsc.addupdate_scatter(accs[2 * j + 1], [d], vhi * wv)

        @pl.loop(0, n_chunks, step=NBUF)
        def _(c):
            for b in range(NBUF):
                cur = c + b
                pltpu.make_async_copy(e_hbm.at[cur], ebufs[b], sems[b]).wait()
                process(ebufs[b])
                nxt = cur + NBUF

                @pl.when(nxt < n_chunks)
                def _():
                    pltpu.make_async_copy(
                        e_hbm.at[nxt], ebufs[b], sems[b]
                    ).start()

        for j in range(C_PER_TILE):
            pltpu.sync_copy(accs[j], o_hbm.at[wid * C_PER_TILE + j])

    return sc_kernel(xt, edges)


def _project(st, w, b):
    d, n = st.shape
    c = w.shape[1]

    def body(st_ref, w_ref, b_ref, o_ref):
        o_ref[...] = (
            lax.dot_general(
                st_ref[...].astype(jnp.bfloat16),
                w_ref[...].astype(jnp.bfloat16),
                dimension_numbers=(((0,), (0,)), ((), ())),
                preferred_element_type=jnp.float32,
            )
            + b_ref[...]
        )

    return pl.pallas_call(
        body,
        out_shape=jax.ShapeDtypeStruct((n, c), jnp.float32),
    )(st, w, b.reshape(1, c))


def kernel(x, edge_index, edge_weight, kernel, bias):
    n_nodes, d_feat = x.shape
    n_edges = edge_index.shape[1]

    dst = edge_index[0].astype(jnp.int32)
    src = edge_index[1].astype(jnp.int32)
    pack = (src << 14) | dst

    n_chunks = -(-n_edges // EDGE_CHUNK)
    n_chunks += n_chunks % NBUF
    e_pad = n_chunks * EDGE_CHUNK
    pad = e_pad - n_edges
    wpad = edge_weight.astype(jnp.float32)
    if pad:
        pack = jnp.pad(pack, (0, pad))
        wpad = jnp.pad(wpad, (0, pad))
    wu = lax.bitcast_convert_type(
        wpad.astype(jnp.bfloat16).reshape(n_chunks, EDGE_CHUNK), jnp.uint16
    ).astype(jnp.uint32)
    wpair = lax.bitcast_convert_type(
        (wu[:, EDGE_CHUNK // 2:] << 16) | wu[:, : EDGE_CHUNK // 2], jnp.int32
    )
    edges = jnp.concatenate(
        [pack.reshape(n_chunks, EDGE_CHUNK), wpair], axis=1
    )

    xu = lax.bitcast_convert_type(
        x.astype(jnp.bfloat16), jnp.uint16
    ).astype(jnp.uint32)
    xt = lax.bitcast_convert_type(
        (xu[:, 1::2] << 16) | xu[:, 0::2], jnp.int32
    ).T

    st = _scatter_accumulate(xt, edges, n_nodes, n_chunks)

    return _project(st, kernel, bias)


# speedup vs baseline: 1.0124x; 1.0124x over previous
import dataclasses
import functools

import jax
import jax.numpy as jnp
from jax import lax
from jax.experimental import pallas as pl
from jax.experimental.pallas import tpu as pltpu
from jax.experimental.pallas import tpu_sc as plsc

N_CORES = 2
N_SUBCORES = 16
N_WORKERS = N_CORES * N_SUBCORES
LANES = 16
C_PER_TILE = 4

EDGE_CHUNK = 16000
NBUF = 2
GROUP_UNROLL = 4


def _scatter_accumulate(xt, edges, n_nodes, n_chunks):
    mesh = plsc.VectorSubcoreMesh(core_axis_name="c", subcore_axis_name="s")
    cp = pltpu.CompilerParams()
    if "needs_layout_passes" in pltpu.CompilerParams.__dataclass_fields__:
        cp = dataclasses.replace(cp, needs_layout_passes=False)

    n_pairs = C_PER_TILE // 2

    @functools.partial(
        pl.kernel,
        compiler_params=cp,
        out_type=jax.ShapeDtypeStruct((N_WORKERS * C_PER_TILE, n_nodes), jnp.float32),
        mesh=mesh,
        scratch_types=(
            [pltpu.VMEM((n_nodes,), jnp.int32) for _ in range(n_pairs)]
            + [pltpu.VMEM((n_nodes,), jnp.float32) for _ in range(C_PER_TILE)]
            + [
                pltpu.VMEM((2, EDGE_CHUNK), jnp.int32),
                pltpu.VMEM((2, EDGE_CHUNK), jnp.int32),
                pltpu.SemaphoreType.DMA,
                pltpu.SemaphoreType.DMA,
            ]
        ),
    )
    def sc_kernel(xt_hbm, e_hbm, o_hbm, xp0, xp1, a0, a1, a2, a3,
                  ebuf0, ebuf1, sem0, sem1):
        ebufs = (ebuf0, ebuf1)
        xps = (xp0, xp1)
        accs = (a0, a1, a2, a3)
        sems = (sem0, sem1)
        wid = lax.axis_index("c") * N_SUBCORES + lax.axis_index("s")
        pair0 = wid * n_pairs

        for j in range(n_pairs):
            pltpu.sync_copy(xt_hbm.at[pair0 + j], xps[j])

        zeros = jnp.zeros((LANES,), jnp.float32)

        @pl.loop(0, n_nodes, step=LANES)
        def _(i):
            for j in range(C_PER_TILE):
                accs[j][pl.ds(i, LANES)] = zeros

        for b in range(NBUF):
            pltpu.make_async_copy(e_hbm.at[b], ebufs[b], sems[b]).start()

        himask = jnp.int32(-65536)

        def process(buf):
            @plsc.parallel_loop(0, EDGE_CHUNK, step=LANES, unroll=GROUP_UNROLL)
            def _(i):
                pk = buf[0, pl.ds(i, LANES)]
                wv = plsc.bitcast(buf[1, pl.ds(i, LANES)], jnp.float32)
                s = pk >> 14
                d = pk & 16383
                for j in range(n_pairs):
                    g = plsc.load_gather(xps[j], [s])
                    vlo = plsc.bitcast(g << 16, jnp.float32)
                    vhi = plsc.bitcast(g & himask, jnp.float32)
                    plsc.addupdate_scatter(accs[2 * j], [d], vlo * wv)
                    plsc.addupdate_scatter(accs[2 * j + 1], [d], vhi * wv)

        @pl.loop(0, n_chunks, step=NBUF)
        def _(c):
            for b in range(NBUF):
                cur = c + b
                pltpu.make_async_copy(e_hbm.at[cur], ebufs[b], sems[b]).wait()
                process(ebufs[b])
                nxt = cur + NBUF

                @pl.when(nxt < n_chunks)
                def _():
                    pltpu.make_async_copy(
                        e_hbm.at[nxt], ebufs[b], sems[b]
                    ).start()

        for j in range(C_PER_TILE):
            pltpu.sync_copy(accs[j], o_hbm.at[wid * C_PER_TILE + j])

    return sc_kernel(xt, edges)


def _project(st, w, b):
    d, n = st.shape
    c = w.shape[1]

    def body(st_ref, w_ref, b_ref, o_ref):
        o_ref[...] = (
            lax.dot_general(
                st_ref[...].astype(jnp.bfloat16),
                w_ref[...].astype(jnp.bfloat16),
                dimension_numbers=(((0,), (0,)), ((), ())),
                preferred_element_type=jnp.float32,
            )
            + b_ref[...]
        )

    return pl.pallas_call(
        body,
        out_shape=jax.ShapeDtypeStruct((n, c), jnp.float32),
    )(st, w, b.reshape(1, c))


def kernel(x, edge_index, edge_weight, kernel, bias):
    n_nodes, d_feat = x.shape
    n_edges = edge_index.shape[1]

    dst = edge_index[0].astype(jnp.int32)
    src = edge_index[1].astype(jnp.int32)
    pack = (src << 14) | dst

    wbits = lax.bitcast_convert_type(edge_weight.astype(jnp.float32), jnp.int32)

    n_chunks = -(-n_edges // EDGE_CHUNK)
    n_chunks += n_chunks % NBUF
    e_pad = n_chunks * EDGE_CHUNK
    pad = e_pad - n_edges
    if pad:
        pack = jnp.pad(pack, (0, pad))
        wbits = jnp.pad(wbits, (0, pad))
    edges = jnp.stack(
        [pack.reshape(n_chunks, EDGE_CHUNK), wbits.reshape(n_chunks, EDGE_CHUNK)],
        axis=1,
    )

    xu = lax.bitcast_convert_type(
        x.astype(jnp.bfloat16), jnp.uint16
    ).astype(jnp.uint32)
    xt = lax.bitcast_convert_type(
        (xu[:, 1::2] << 16) | xu[:, 0::2], jnp.int32
    ).T

    st = _scatter_accumulate(xt, edges, n_nodes, n_chunks)

    return _project(st, kernel, bias)
